# baseline (speedup 1.0000x reference)
import jax
import jax.numpy as jnp
from jax import lax
from jax.experimental import pallas as pl
from jax.experimental.pallas import tpu as pltpu
from jax.experimental.pallas import tpu_sc as plsc

N = 10000
D = 128
E = 320000
NC, NS = 2, 16
NT = NC * NS
CH = 128
NCH = 80
NPH = 2
HCH = NCH // NPH
EPT = CH * NCH
E_PAD = EPT * NT
N_ACC = 10240
RPT = N_ACC // NS
DUMP = N
BR = 400
NB = N // BR


def _sc_agg_body(h_hbm, src_hbm, dst_hbm, zeros_hbm, out_hbm,
                 src_v, dst_v, buf_0, buf_1, accum,
                 gs_0, gs_1, ss_0, ss_1):
    cid = lax.axis_index("c")
    sid = lax.axis_index("s")
    wid = cid * NS + sid
    bufs = (buf_0, buf_1)
    gsem = (gs_0, gs_1)
    ssem = (ss_0, ss_1)
    pltpu.sync_copy(zeros_hbm, accum.at[pl.ds(sid * RPT, RPT)])
    plsc.subcore_barrier()

    def _gather_start(c, j):
        pltpu.make_async_copy(h_hbm.at[src_v.at[c]], bufs[j], gsem[j]).start()

    def _gather_wait(c, j):
        pltpu.make_async_copy(h_hbm.at[src_v.at[c]], bufs[j], gsem[j]).wait()

    def _scatter_start(c, j):
        pltpu.make_async_copy(bufs[j], accum.at[dst_v.at[c]],
                              ssem[j]).start(add=True)

    def _scatter_wait(c, j):
        pltpu.make_async_copy(bufs[j], accum.at[dst_v.at[c]], ssem[j]).wait()

    @pl.loop(0, NPH)
    def _phase(p):
        pltpu.sync_copy(src_hbm.at[wid, pl.ds(p * HCH, HCH)], src_v)
        pltpu.sync_copy(dst_hbm.at[wid, pl.ds(p * HCH, HCH)], dst_v)
        _gather_start(0, 0)
        _gather_start(1, 1)

        @pl.loop(0, HCH, step=2)
        def _chunk(g):
            _gather_wait(g, 0)
            _scatter_start(g, 0)
            _gather_wait(g + 1, 1)
            _scatter_start(g + 1, 1)
            _scatter_wait(g, 0)

            @pl.when(g + 2 < HCH)
            def _():
                _gather_start(g + 2, 0)

            _scatter_wait(g + 1, 1)

            @pl.when(g + 3 < HCH)
            def _():
                _gather_start(g + 3, 1)

    plsc.subcore_barrier()
    pltpu.sync_copy(accum.at[pl.ds(sid * RPT, RPT)],
                    out_hbm.at[cid, pl.ds(sid * RPT, RPT)])


_sc_agg = pl.kernel(
    _sc_agg_body,
    out_type=jax.ShapeDtypeStruct((NC, N_ACC, D), jnp.float32),
    mesh=plsc.VectorSubcoreMesh(core_axis_name="c", subcore_axis_name="s",
                                num_cores=NC, num_subcores=NS),
    scratch_types=[
        pltpu.VMEM((HCH, CH), jnp.int32),
        pltpu.VMEM((HCH, CH), jnp.int32),
        pltpu.VMEM((CH, D), jnp.float32),
        pltpu.VMEM((CH, D), jnp.float32),
        pltpu.VMEM_SHARED((N_ACC, D), jnp.float32),
        pltpu.SemaphoreType.DMA,
        pltpu.SemaphoreType.DMA,
        pltpu.SemaphoreType.DMA,
        pltpu.SemaphoreType.DMA,
    ],
)


def _mm_bias_body(x_ref, w_ref, b_ref, o_ref):
    o_ref[...] = lax.dot_general(
        x_ref[...], w_ref[...], (((1,), (1,)), ((), ())),
        preferred_element_type=jnp.float32) + b_ref[...]


def _agg_mm_body(p_ref, w_ref, b_ref, o_ref):
    a = jnp.maximum(p_ref[0] + p_ref[1], 0.0)
    o_ref[...] = lax.dot_general(
        a, w_ref[...], (((1,), (1,)), ((), ())),
        preferred_element_type=jnp.float32) + b_ref[...]


def _relu_agg_body(p_ref, o_ref):
    o_ref[...] = jnp.maximum(p_ref[0] + p_ref[1], 0.0)


_w_spec = pl.BlockSpec((D, D), lambda i: (0, 0))
_b_spec = pl.BlockSpec((1, D), lambda i: (0, 0))
_row_spec = pl.BlockSpec((BR, D), lambda i: (i, 0))
_p_spec = pl.BlockSpec((NC, BR, D), lambda i: (0, i, 0))

_mm_bias = pl.pallas_call(
    _mm_bias_body,
    grid=(NB,),
    in_specs=[_row_spec, _w_spec, _b_spec],
    out_specs=_row_spec,
    out_shape=jax.ShapeDtypeStruct((N, D), jnp.float32),
)

_agg_mm = pl.pallas_call(
    _agg_mm_body,
    grid=(NB,),
    in_specs=[_p_spec, _w_spec, _b_spec],
    out_specs=_row_spec,
    out_shape=jax.ShapeDtypeStruct((N, D), jnp.float32),
)

_relu_agg = pl.pallas_call(
    _relu_agg_body,
    grid=(NB,),
    in_specs=[_p_spec],
    out_specs=_row_spec,
    out_shape=jax.ShapeDtypeStruct((N, D), jnp.float32),
)


def kernel(x, edge_index, W0, b0, W1, b1):
    ept_real = E // NT
    padt = EPT - ept_real
    src = edge_index[0].astype(jnp.int32).reshape(NT, ept_real)
    dst = edge_index[1].astype(jnp.int32).reshape(NT, ept_real)
    pad_src = jnp.broadcast_to(jnp.arange(padt, dtype=jnp.int32), (NT, padt))
    pad_dst = jnp.broadcast_to(DUMP + jnp.arange(padt, dtype=jnp.int32), (NT, padt))
    src = jnp.concatenate([src, pad_src], axis=1).reshape(NT, NCH, CH)
    dst = jnp.concatenate([dst, pad_dst], axis=1).reshape(NT, NCH, CH)
    zeros = jnp.zeros((RPT, D), jnp.float32)
    b0r = b0.reshape(1, D)
    b1r = b1.reshape(1, D)

    h1 = _mm_bias(x, W0, b0r)
    p = _sc_agg(h1, src, dst, zeros)
    h2 = _agg_mm(p, W1, b1r)
    q = _sc_agg(h2, src, dst, zeros)
    return _relu_agg(q)


# speedup vs baseline: 1.2369x; 1.2369x over previous
import jax
import jax.numpy as jnp
from jax import lax
from jax.experimental import pallas as pl
from jax.experimental.pallas import tpu as pltpu
from jax.experimental.pallas import tpu_sc as plsc

N = 10000
D = 128
E = 320000
NC, NS = 2, 16
NT = NC * NS
CH = 128
NCH = 80
NPH = 2
HCH = NCH // NPH
EPT = CH * NCH
E_PAD = EPT * NT
N_ACC = 10240
RPT = N_ACC // NS
DUMP = N
BR = 400
NB = N // BR


def _sc_agg_body(h_hbm, src_hbm, dst_hbm, zeros_hbm, out_hbm,
                 src_v, dst_v, buf_0, buf_1, accum, gs_0, gs_1):
    cid = lax.axis_index("c")
    sid = lax.axis_index("s")
    wid = cid * NS + sid
    bufs = (buf_0, buf_1)
    gsem = (gs_0, gs_1)
    pltpu.sync_copy(zeros_hbm, accum.at[pl.ds(sid * RPT, RPT)])
    plsc.subcore_barrier()

    def _gather_start(c, j):
        pltpu.make_async_copy(h_hbm.at[src_v.at[c]], bufs[j], gsem[j]).start()

    def _gather_wait(c, j):
        pltpu.make_async_copy(h_hbm.at[src_v.at[c]], bufs[j], gsem[j]).wait()

    @pl.loop(0, NPH)
    def _phase(p):
        pltpu.sync_copy(src_hbm.at[wid, pl.ds(p * HCH, HCH)], src_v)
        pltpu.sync_copy(dst_hbm.at[wid, pl.ds(p * HCH, HCH)], dst_v)
        _gather_start(0, 0)

        @pl.loop(0, HCH, step=2)
        def _chunk(g):
            _gather_start(g + 1, 1)
            _gather_wait(g, 0)
            pltpu.sync_copy(bufs[0], accum.at[dst_v.at[g]], add=True)

            @pl.when(g + 2 < HCH)
            def _():
                _gather_start(g + 2, 0)

            _gather_wait(g + 1, 1)
            pltpu.sync_copy(bufs[1], accum.at[dst_v.at[g + 1]], add=True)

    plsc.subcore_barrier()
    pltpu.sync_copy(accum.at[pl.ds(sid * RPT, RPT)],
                    out_hbm.at[cid, pl.ds(sid * RPT, RPT)])


_sc_agg = pl.kernel(
    _sc_agg_body,
    out_type=jax.ShapeDtypeStruct((NC, N_ACC, D), jnp.float32),
    mesh=plsc.VectorSubcoreMesh(core_axis_name="c", subcore_axis_name="s",
                                num_cores=NC, num_subcores=NS),
    scratch_types=[
        pltpu.VMEM((HCH, CH), jnp.int32),
        pltpu.VMEM((HCH, CH), jnp.int32),
        pltpu.VMEM((CH, D), jnp.float32),
        pltpu.VMEM((CH, D), jnp.float32),
        pltpu.VMEM_SHARED((N_ACC, D), jnp.float32),
        pltpu.SemaphoreType.DMA,
        pltpu.SemaphoreType.DMA,
    ],
)


def _mm_bias_body(x_ref, w_ref, b_ref, o_ref):
    o_ref[...] = lax.dot_general(
        x_ref[...], w_ref[...], (((1,), (1,)), ((), ())),
        preferred_element_type=jnp.float32) + b_ref[...]


def _agg_mm_body(p_ref, w_ref, b_ref, o_ref):
    a = jnp.maximum(p_ref[0] + p_ref[1], 0.0)
    o_ref[...] = lax.dot_general(
        a, w_ref[...], (((1,), (1,)), ((), ())),
        preferred_element_type=jnp.float32) + b_ref[...]


def _relu_agg_body(p_ref, o_ref):
    o_ref[...] = jnp.maximum(p_ref[0] + p_ref[1], 0.0)


_w_spec = pl.BlockSpec((D, D), lambda i: (0, 0))
_b_spec = pl.BlockSpec((1, D), lambda i: (0, 0))
_row_spec = pl.BlockSpec((BR, D), lambda i: (i, 0))
_p_spec = pl.BlockSpec((NC, BR, D), lambda i: (0, i, 0))

_mm_bias = pl.pallas_call(
    _mm_bias_body,
    grid=(NB,),
    in_specs=[_row_spec, _w_spec, _b_spec],
    out_specs=_row_spec,
    out_shape=jax.ShapeDtypeStruct((N, D), jnp.float32),
)

_agg_mm = pl.pallas_call(
    _agg_mm_body,
    grid=(NB,),
    in_specs=[_p_spec, _w_spec, _b_spec],
    out_specs=_row_spec,
    out_shape=jax.ShapeDtypeStruct((N, D), jnp.float32),
)

_relu_agg = pl.pallas_call(
    _relu_agg_body,
    grid=(NB,),
    in_specs=[_p_spec],
    out_specs=_row_spec,
    out_shape=jax.ShapeDtypeStruct((N, D), jnp.float32),
)


def kernel(x, edge_index, W0, b0, W1, b1):
    ept_real = E // NT
    padt = EPT - ept_real
    src = edge_index[0].astype(jnp.int32).reshape(NT, ept_real)
    dst = edge_index[1].astype(jnp.int32).reshape(NT, ept_real)
    pad_src = jnp.broadcast_to(jnp.arange(padt, dtype=jnp.int32), (NT, padt))
    pad_dst = jnp.broadcast_to(DUMP + jnp.arange(padt, dtype=jnp.int32), (NT, padt))
    src = jnp.concatenate([src, pad_src], axis=1).reshape(NT, NCH, CH)
    dst = jnp.concatenate([dst, pad_dst], axis=1).reshape(NT, NCH, CH)
    zeros = jnp.zeros((RPT, D), jnp.float32)
    b0r = b0.reshape(1, D)
    b1r = b1.reshape(1, D)

    h1 = _mm_bias(x, W0, b0r)
    p = _sc_agg(h1, src, dst, zeros)
    h2 = _agg_mm(p, W1, b1r)
    q = _sc_agg(h2, src, dst, zeros)
    return _relu_agg(q)


# speedup vs baseline: 1.2491x; 1.0098x over previous
import jax
import jax.numpy as jnp
from jax import lax
from jax.experimental import pallas as pl
from jax.experimental.pallas import tpu as pltpu
from jax.experimental.pallas import tpu_sc as plsc

N = 10000
D = 128
E = 320000
NC, NS = 2, 16
NT = NC * NS
CH = 128
NCH = 80
NPH = 2
HCH = NCH // NPH
EPT = CH * NCH
E_PAD = EPT * NT
N_ACC = 10240
RPT = N_ACC // NS
DUMP = N
BR = 400
NB = N // BR


def _sc_agg_body(h_hbm, src_hbm, dst_hbm, zeros_hbm, out_hbm,
                 src_v, dst_v, buf_0, buf_1, accum, gs_0, gs_1):
    cid = lax.axis_index("c")
    sid = lax.axis_index("s")
    wid = cid * NS + sid
    bufs = (buf_0, buf_1)
    gsem = (gs_0, gs_1)

    def _gather_start(c, j):
        pltpu.make_async_copy(h_hbm.at[src_v.at[c]], bufs[j], gsem[j]).start()

    def _gather_wait(c, j):
        pltpu.make_async_copy(h_hbm.at[src_v.at[c]], bufs[j], gsem[j]).wait()

    def _inner(g):
        _gather_start(g + 1, 1)
        _gather_wait(g, 0)
        pltpu.sync_copy(bufs[0], accum.at[dst_v.at[g]], add=True)

        @pl.when(g + 2 < HCH)
        def _():
            _gather_start(g + 2, 0)

        _gather_wait(g + 1, 1)
        pltpu.sync_copy(bufs[1], accum.at[dst_v.at[g + 1]], add=True)

    pltpu.sync_copy(src_hbm.at[wid, pl.ds(0, HCH)], src_v)
    _gather_start(0, 0)
    pltpu.sync_copy(dst_hbm.at[wid, pl.ds(0, HCH)], dst_v)
    pltpu.sync_copy(zeros_hbm, accum.at[pl.ds(sid * RPT, RPT)])
    plsc.subcore_barrier()

    for p in range(NPH):
        if p > 0:
            pltpu.sync_copy(src_hbm.at[wid, pl.ds(p * HCH, HCH)], src_v)
            pltpu.sync_copy(dst_hbm.at[wid, pl.ds(p * HCH, HCH)], dst_v)
            _gather_start(0, 0)
        pl.loop(0, HCH, step=2)(_inner)

    plsc.subcore_barrier()
    pltpu.sync_copy(accum.at[pl.ds(sid * RPT, RPT)],
                    out_hbm.at[cid, pl.ds(sid * RPT, RPT)])


_sc_agg = pl.kernel(
    _sc_agg_body,
    out_type=jax.ShapeDtypeStruct((NC, N_ACC, D), jnp.float32),
    mesh=plsc.VectorSubcoreMesh(core_axis_name="c", subcore_axis_name="s",
                                num_cores=NC, num_subcores=NS),
    scratch_types=[
        pltpu.VMEM((HCH, CH), jnp.int32),
        pltpu.VMEM((HCH, CH), jnp.int32),
        pltpu.VMEM((CH, D), jnp.float32),
        pltpu.VMEM((CH, D), jnp.float32),
        pltpu.VMEM_SHARED((N_ACC, D), jnp.float32),
        pltpu.SemaphoreType.DMA,
        pltpu.SemaphoreType.DMA,
    ],
)


def _mm_bias_body(x_ref, w_ref, b_ref, o_ref):
    o_ref[...] = lax.dot_general(
        x_ref[...], w_ref[...], (((1,), (1,)), ((), ())),
        preferred_element_type=jnp.float32) + b_ref[...]


def _agg_mm_body(p_ref, w_ref, b_ref, o_ref):
    a = jnp.maximum(p_ref[0] + p_ref[1], 0.0)
    o_ref[...] = lax.dot_general(
        a, w_ref[...], (((1,), (1,)), ((), ())),
        preferred_element_type=jnp.float32) + b_ref[...]


def _relu_agg_body(p_ref, o_ref):
    o_ref[...] = jnp.maximum(p_ref[0] + p_ref[1], 0.0)


_w_spec = pl.BlockSpec((D, D), lambda i: (0, 0))
_b_spec = pl.BlockSpec((1, D), lambda i: (0, 0))
_row_spec = pl.BlockSpec((BR, D), lambda i: (i, 0))
_p_spec = pl.BlockSpec((NC, BR, D), lambda i: (0, i, 0))

_mm_bias = pl.pallas_call(
    _mm_bias_body,
    grid=(NB,),
    in_specs=[_row_spec, _w_spec, _b_spec],
    out_specs=_row_spec,
    out_shape=jax.ShapeDtypeStruct((N, D), jnp.float32),
)

_agg_mm = pl.pallas_call(
    _agg_mm_body,
    grid=(NB,),
    in_specs=[_p_spec, _w_spec, _b_spec],
    out_specs=_row_spec,
    out_shape=jax.ShapeDtypeStruct((N, D), jnp.float32),
)

_relu_agg = pl.pallas_call(
    _relu_agg_body,
    grid=(NB,),
    in_specs=[_p_spec],
    out_specs=_row_spec,
    out_shape=jax.ShapeDtypeStruct((N, D), jnp.float32),
)


def kernel(x, edge_index, W0, b0, W1, b1):
    ept_real = E // NT
    padt = EPT - ept_real
    src = edge_index[0].astype(jnp.int32).reshape(NT, ept_real)
    dst = edge_index[1].astype(jnp.int32).reshape(NT, ept_real)
    pad_src = jnp.broadcast_to(jnp.arange(padt, dtype=jnp.int32), (NT, padt))
    pad_dst = jnp.broadcast_to(DUMP + jnp.arange(padt, dtype=jnp.int32), (NT, padt))
    src = jnp.concatenate([src, pad_src], axis=1).reshape(NT, NCH, CH)
    dst = jnp.concatenate([dst, pad_dst], axis=1).reshape(NT, NCH, CH)
    zeros = jnp.zeros((RPT, D), jnp.float32)
    b0r = b0.reshape(1, D)
    b1r = b1.reshape(1, D)

    h1 = _mm_bias(x, W0, b0r)
    p = _sc_agg(h1, src, dst, zeros)
    h2 = _agg_mm(p, W1, b1r)
    q = _sc_agg(h2, src, dst, zeros)
    return _relu_agg(q)


# speedup vs baseline: 1.3443x; 1.0762x over previous
import jax
import jax.numpy as jnp
from jax import lax
from jax.experimental import pallas as pl
from jax.experimental.pallas import tpu as pltpu
from jax.experimental.pallas import tpu_sc as plsc

N = 10000
D = 128
E = 320000
NC, NS = 2, 16
NT = NC * NS
CH = 128
NCH = 80
NPH = 2
HCH = NCH // NPH
EPT = CH * NCH
E_PAD = EPT * NT
N_ACC = 10240
RPT = N_ACC // NS
DUMP = N
BR = 1000
NB = N // BR


def _sc_agg_body(h_hbm, src_hbm, dst_hbm, zeros_hbm, out_hbm,
                 src_v, dst_v, buf_0, buf_1, accum, gs_0, gs_1):
    cid = lax.axis_index("c")
    sid = lax.axis_index("s")
    wid = cid * NS + sid
    bufs = (buf_0, buf_1)
    gsem = (gs_0, gs_1)

    def _gather_start(c, j):
        pltpu.make_async_copy(h_hbm.at[src_v.at[c]], bufs[j], gsem[j]).start()

    def _gather_wait(c, j):
        pltpu.make_async_copy(h_hbm.at[src_v.at[c]], bufs[j], gsem[j]).wait()

    def _inner(g):
        _gather_start(g + 1, 1)
        _gather_wait(g, 0)
        pltpu.sync_copy(bufs[0], accum.at[dst_v.at[g]], add=True)

        @pl.when(g + 2 < HCH)
        def _():
            _gather_start(g + 2, 0)

        _gather_wait(g + 1, 1)
        pltpu.sync_copy(bufs[1], accum.at[dst_v.at[g + 1]], add=True)

    pltpu.sync_copy(src_hbm.at[wid, pl.ds(0, HCH)], src_v)
    _gather_start(0, 0)
    pltpu.sync_copy(dst_hbm.at[wid, pl.ds(0, HCH)], dst_v)
    pltpu.sync_copy(zeros_hbm, accum.at[pl.ds(sid * RPT, RPT)])
    plsc.subcore_barrier()

    for p in range(NPH):
        if p > 0:
            pltpu.sync_copy(src_hbm.at[wid, pl.ds(p * HCH, HCH)], src_v)
            pltpu.sync_copy(dst_hbm.at[wid, pl.ds(p * HCH, HCH)], dst_v)
            _gather_start(0, 0)
        pl.loop(0, HCH, step=2)(_inner)

    plsc.subcore_barrier()
    pltpu.sync_copy(accum.at[pl.ds(sid * RPT, RPT)],
                    out_hbm.at[cid, pl.ds(sid * RPT, RPT)])


_sc_agg = pl.kernel(
    _sc_agg_body,
    out_type=jax.ShapeDtypeStruct((NC, N_ACC, D), jnp.float32),
    mesh=plsc.VectorSubcoreMesh(core_axis_name="c", subcore_axis_name="s",
                                num_cores=NC, num_subcores=NS),
    scratch_types=[
        pltpu.VMEM((HCH, CH), jnp.int32),
        pltpu.VMEM((HCH, CH), jnp.int32),
        pltpu.VMEM((CH, D), jnp.float32),
        pltpu.VMEM((CH, D), jnp.float32),
        pltpu.VMEM_SHARED((N_ACC, D), jnp.float32),
        pltpu.SemaphoreType.DMA,
        pltpu.SemaphoreType.DMA,
    ],
)


def _mm_bias_body(x_ref, w_ref, b_ref, o_ref):
    o_ref[...] = lax.dot_general(
        x_ref[...], w_ref[...], (((1,), (1,)), ((), ())),
        preferred_element_type=jnp.float32) + b_ref[...]


def _agg_mm_body(p_ref, w_ref, b_ref, o_ref):
    a = jnp.maximum(p_ref[0] + p_ref[1], 0.0)
    o_ref[...] = lax.dot_general(
        a, w_ref[...], (((1,), (1,)), ((), ())),
        preferred_element_type=jnp.float32) + b_ref[...]


def _relu_agg_body(p_ref, o_ref):
    o_ref[...] = jnp.maximum(p_ref[0] + p_ref[1], 0.0)


_w_spec = pl.BlockSpec((D, D), lambda i: (0, 0))
_b_spec = pl.BlockSpec((1, D), lambda i: (0, 0))
_row_spec = pl.BlockSpec((BR, D), lambda i: (i, 0))
_p_spec = pl.BlockSpec((NC, BR, D), lambda i: (0, i, 0))

_mm_bias = pl.pallas_call(
    _mm_bias_body,
    grid=(NB,),
    in_specs=[_row_spec, _w_spec, _b_spec],
    out_specs=_row_spec,
    out_shape=jax.ShapeDtypeStruct((N, D), jnp.float32),
)

_agg_mm = pl.pallas_call(
    _agg_mm_body,
    grid=(NB,),
    in_specs=[_p_spec, _w_spec, _b_spec],
    out_specs=_row_spec,
    out_shape=jax.ShapeDtypeStruct((N, D), jnp.float32),
)

_relu_agg = pl.pallas_call(
    _relu_agg_body,
    grid=(NB,),
    in_specs=[_p_spec],
    out_specs=_row_spec,
    out_shape=jax.ShapeDtypeStruct((N, D), jnp.float32),
)


def kernel(x, edge_index, W0, b0, W1, b1):
    ept_real = E // NT
    padt = EPT - ept_real
    src = edge_index[0].astype(jnp.int32).reshape(NT, ept_real)
    dst = edge_index[1].astype(jnp.int32).reshape(NT, ept_real)
    pad_src = jnp.broadcast_to(jnp.arange(padt, dtype=jnp.int32), (NT, padt))
    pad_dst = jnp.broadcast_to(DUMP + jnp.arange(padt, dtype=jnp.int32), (NT, padt))
    src = jnp.concatenate([src, pad_src], axis=1).reshape(NT, NCH, CH)
    dst = jnp.concatenate([dst, pad_dst], axis=1).reshape(NT, NCH, CH)
    zeros = jnp.zeros((RPT, D), jnp.float32)
    b0r = b0.reshape(1, D)
    b1r = b1.reshape(1, D)

    h1 = _mm_bias(x, W0, b0r)
    p = _sc_agg(h1, src, dst, zeros)
    h2 = _agg_mm(p, W1, b1r)
    q = _sc_agg(h2, src, dst, zeros)
    return _relu_agg(q)


# speedup vs baseline: 1.3800x; 1.0266x over previous
import jax
import jax.numpy as jnp
from jax import lax
from jax.experimental import pallas as pl
from jax.experimental.pallas import tpu as pltpu
from jax.experimental.pallas import tpu_sc as plsc

N = 10000
D = 128
E = 320000
NC, NS = 2, 16
NT = NC * NS
CH = 128
NCH = 80
NPH = 2
HCH = NCH // NPH
EPT = CH * NCH
E_PAD = EPT * NT
N_ACC = 10240
RPT = N_ACC // NS
DUMP = N
BR = 2000
NB = N // BR


def _sc_agg_body(h_hbm, src_hbm, dst_hbm, zeros_hbm, out_hbm,
                 src_v, dst_v, buf_0, buf_1, accum, gs_0, gs_1):
    cid = lax.axis_index("c")
    sid = lax.axis_index("s")
    wid = cid * NS + sid
    bufs = (buf_0, buf_1)
    gsem = (gs_0, gs_1)

    def _gather_start(c, j):
        pltpu.make_async_copy(h_hbm.at[src_v.at[c]], bufs[j], gsem[j]).start()

    def _gather_wait(c, j):
        pltpu.make_async_copy(h_hbm.at[src_v.at[c]], bufs[j], gsem[j]).wait()

    def _inner(g):
        _gather_start(g + 1, 1)
        _gather_wait(g, 0)
        pltpu.sync_copy(bufs[0], accum.at[dst_v.at[g]], add=True)

        @pl.when(g + 2 < HCH)
        def _():
            _gather_start(g + 2, 0)

        _gather_wait(g + 1, 1)
        pltpu.sync_copy(bufs[1], accum.at[dst_v.at[g + 1]], add=True)

    pltpu.sync_copy(src_hbm.at[wid, pl.ds(0, HCH)], src_v)
    _gather_start(0, 0)
    pltpu.sync_copy(dst_hbm.at[wid, pl.ds(0, HCH)], dst_v)
    pltpu.sync_copy(zeros_hbm, accum.at[pl.ds(sid * RPT, RPT)])
    plsc.subcore_barrier()

    for p in range(NPH):
        if p > 0:
            pltpu.sync_copy(src_hbm.at[wid, pl.ds(p * HCH, HCH)], src_v)
            pltpu.sync_copy(dst_hbm.at[wid, pl.ds(p * HCH, HCH)], dst_v)
            _gather_start(0, 0)
        pl.loop(0, HCH, step=2)(_inner)

    plsc.subcore_barrier()
    pltpu.sync_copy(accum.at[pl.ds(sid * RPT, RPT)],
                    out_hbm.at[cid, pl.ds(sid * RPT, RPT)])


_sc_agg = pl.kernel(
    _sc_agg_body,
    out_type=jax.ShapeDtypeStruct((NC, N_ACC, D), jnp.float32),
    mesh=plsc.VectorSubcoreMesh(core_axis_name="c", subcore_axis_name="s",
                                num_cores=NC, num_subcores=NS),
    scratch_types=[
        pltpu.VMEM((HCH, CH), jnp.int32),
        pltpu.VMEM((HCH, CH), jnp.int32),
        pltpu.VMEM((CH, D), jnp.float32),
        pltpu.VMEM((CH, D), jnp.float32),
        pltpu.VMEM_SHARED((N_ACC, D), jnp.float32),
        pltpu.SemaphoreType.DMA,
        pltpu.SemaphoreType.DMA,
    ],
)


def _mm_bias_body(x_ref, w_ref, b_ref, o_ref):
    o_ref[...] = lax.dot_general(
        x_ref[...], w_ref[...], (((1,), (1,)), ((), ())),
        preferred_element_type=jnp.float32) + b_ref[...]


def _agg_mm_body(p_ref, w_ref, b_ref, o_ref):
    a = jnp.maximum(p_ref[0] + p_ref[1], 0.0)
    o_ref[...] = lax.dot_general(
        a, w_ref[...], (((1,), (1,)), ((), ())),
        preferred_element_type=jnp.float32) + b_ref[...]


def _relu_agg_body(p_ref, o_ref):
    o_ref[...] = jnp.maximum(p_ref[0] + p_ref[1], 0.0)


_w_spec = pl.BlockSpec((D, D), lambda i: (0, 0))
_b_spec = pl.BlockSpec((1, D), lambda i: (0, 0))
_row_spec = pl.BlockSpec((BR, D), lambda i: (i, 0))
_p_spec = pl.BlockSpec((NC, BR, D), lambda i: (0, i, 0))

_mm_bias = pl.pallas_call(
    _mm_bias_body,
    grid=(NB,),
    in_specs=[_row_spec, _w_spec, _b_spec],
    out_specs=_row_spec,
    out_shape=jax.ShapeDtypeStruct((N, D), jnp.float32),
)

_agg_mm = pl.pallas_call(
    _agg_mm_body,
    grid=(NB,),
    in_specs=[_p_spec, _w_spec, _b_spec],
    out_specs=_row_spec,
    out_shape=jax.ShapeDtypeStruct((N, D), jnp.float32),
)

_relu_agg = pl.pallas_call(
    _relu_agg_body,
    grid=(NB,),
    in_specs=[_p_spec],
    out_specs=_row_spec,
    out_shape=jax.ShapeDtypeStruct((N, D), jnp.float32),
)


def kernel(x, edge_index, W0, b0, W1, b1):
    ept_real = E // NT
    padt = EPT - ept_real
    src = edge_index[0].astype(jnp.int32).reshape(NT, ept_real)
    dst = edge_index[1].astype(jnp.int32).reshape(NT, ept_real)
    pad_src = jnp.broadcast_to(jnp.arange(padt, dtype=jnp.int32), (NT, padt))
    pad_dst = jnp.broadcast_to(DUMP + jnp.arange(padt, dtype=jnp.int32), (NT, padt))
    src = jnp.concatenate([src, pad_src], axis=1).reshape(NT, NCH, CH)
    dst = jnp.concatenate([dst, pad_dst], axis=1).reshape(NT, NCH, CH)
    zeros = jnp.zeros((RPT, D), jnp.float32)
    b0r = b0.reshape(1, D)
    b1r = b1.reshape(1, D)

    h1 = _mm_bias(x, W0, b0r)
    p = _sc_agg(h1, src, dst, zeros)
    h2 = _agg_mm(p, W1, b1r)
    q = _sc_agg(h2, src, dst, zeros)
    return _relu_agg(q)


# speedup vs baseline: 1.3997x; 1.0142x over previous
import jax
import jax.numpy as jnp
from jax import lax
from jax.experimental import pallas as pl
from jax.experimental.pallas import tpu as pltpu
from jax.experimental.pallas import tpu_sc as plsc

N = 10000
D = 128
E = 320000
NC, NS = 2, 16
NT = NC * NS
CH = 128
NCH = 80
NPH = 2
HCH = NCH // NPH
EPT = CH * NCH
E_PAD = EPT * NT
N_ACC = 10240
RPT = N_ACC // NS
DUMP = N
BR = 10000
NB = N // BR


def _sc_agg_body(h_hbm, src_hbm, dst_hbm, zeros_hbm, out_hbm,
                 src_v, dst_v, buf_0, buf_1, accum, gs_0, gs_1):
    cid = lax.axis_index("c")
    sid = lax.axis_index("s")
    wid = cid * NS + sid
    bufs = (buf_0, buf_1)
    gsem = (gs_0, gs_1)

    def _gather_start(c, j):
        pltpu.make_async_copy(h_hbm.at[src_v.at[c]], bufs[j], gsem[j]).start()

    def _gather_wait(c, j):
        pltpu.make_async_copy(h_hbm.at[src_v.at[c]], bufs[j], gsem[j]).wait()

    def _inner(g):
        _gather_start(g + 1, 1)
        _gather_wait(g, 0)
        pltpu.sync_copy(bufs[0], accum.at[dst_v.at[g]], add=True)

        @pl.when(g + 2 < HCH)
        def _():
            _gather_start(g + 2, 0)

        _gather_wait(g + 1, 1)
        pltpu.sync_copy(bufs[1], accum.at[dst_v.at[g + 1]], add=True)

    pltpu.sync_copy(src_hbm.at[wid, pl.ds(0, HCH)], src_v)
    _gather_start(0, 0)
    pltpu.sync_copy(dst_hbm.at[wid, pl.ds(0, HCH)], dst_v)
    pltpu.sync_copy(zeros_hbm, accum.at[pl.ds(sid * RPT, RPT)])
    plsc.subcore_barrier()

    for p in range(NPH):
        if p > 0:
            pltpu.sync_copy(src_hbm.at[wid, pl.ds(p * HCH, HCH)], src_v)
            pltpu.sync_copy(dst_hbm.at[wid, pl.ds(p * HCH, HCH)], dst_v)
            _gather_start(0, 0)
        pl.loop(0, HCH, step=2)(_inner)

    plsc.subcore_barrier()
    pltpu.sync_copy(accum.at[pl.ds(sid * RPT, RPT)],
                    out_hbm.at[cid, pl.ds(sid * RPT, RPT)])


_sc_agg = pl.kernel(
    _sc_agg_body,
    out_type=jax.ShapeDtypeStruct((NC, N_ACC, D), jnp.float32),
    mesh=plsc.VectorSubcoreMesh(core_axis_name="c", subcore_axis_name="s",
                                num_cores=NC, num_subcores=NS),
    scratch_types=[
        pltpu.VMEM((HCH, CH), jnp.int32),
        pltpu.VMEM((HCH, CH), jnp.int32),
        pltpu.VMEM((CH, D), jnp.float32),
        pltpu.VMEM((CH, D), jnp.float32),
        pltpu.VMEM_SHARED((N_ACC, D), jnp.float32),
        pltpu.SemaphoreType.DMA,
        pltpu.SemaphoreType.DMA,
    ],
)


def _mm_bias_body(x_ref, w_ref, b_ref, o_ref):
    o_ref[...] = lax.dot_general(
        x_ref[...], w_ref[...], (((1,), (1,)), ((), ())),
        preferred_element_type=jnp.float32) + b_ref[...]


def _agg_mm_body(p_ref, w_ref, b_ref, o_ref):
    a = jnp.maximum(p_ref[0] + p_ref[1], 0.0)
    o_ref[...] = lax.dot_general(
        a, w_ref[...], (((1,), (1,)), ((), ())),
        preferred_element_type=jnp.float32) + b_ref[...]


def _relu_agg_body(p_ref, o_ref):
    o_ref[...] = jnp.maximum(p_ref[0] + p_ref[1], 0.0)


_w_spec = pl.BlockSpec((D, D), lambda i: (0, 0))
_b_spec = pl.BlockSpec((1, D), lambda i: (0, 0))
_row_spec = pl.BlockSpec((BR, D), lambda i: (i, 0))
_p_spec = pl.BlockSpec((NC, BR, D), lambda i: (0, i, 0))

_mm_bias = pl.pallas_call(
    _mm_bias_body,
    grid=(NB,),
    in_specs=[_row_spec, _w_spec, _b_spec],
    out_specs=_row_spec,
    out_shape=jax.ShapeDtypeStruct((N, D), jnp.float32),
)

_agg_mm = pl.pallas_call(
    _agg_mm_body,
    grid=(NB,),
    in_specs=[_p_spec, _w_spec, _b_spec],
    out_specs=_row_spec,
    out_shape=jax.ShapeDtypeStruct((N, D), jnp.float32),
)

_relu_agg = pl.pallas_call(
    _relu_agg_body,
    grid=(NB,),
    in_specs=[_p_spec],
    out_specs=_row_spec,
    out_shape=jax.ShapeDtypeStruct((N, D), jnp.float32),
)


def kernel(x, edge_index, W0, b0, W1, b1):
    ept_real = E // NT
    padt = EPT - ept_real
    src = edge_index[0].astype(jnp.int32).reshape(NT, ept_real)
    dst = edge_index[1].astype(jnp.int32).reshape(NT, ept_real)
    pad_src = jnp.broadcast_to(jnp.arange(padt, dtype=jnp.int32), (NT, padt))
    pad_dst = jnp.broadcast_to(DUMP + jnp.arange(padt, dtype=jnp.int32), (NT, padt))
    src = jnp.concatenate([src, pad_src], axis=1).reshape(NT, NCH, CH)
    dst = jnp.concatenate([dst, pad_dst], axis=1).reshape(NT, NCH, CH)
    zeros = jnp.zeros((RPT, D), jnp.float32)
    b0r = b0.reshape(1, D)
    b1r = b1.reshape(1, D)

    h1 = _mm_bias(x, W0, b0r)
    p = _sc_agg(h1, src, dst, zeros)
    h2 = _agg_mm(p, W1, b1r)
    q = _sc_agg(h2, src, dst, zeros)
    return _relu_agg(q)
